# baseline (speedup 1.0000x reference)
import functools

import jax
import jax.numpy as jnp
from jax import lax
from jax.experimental import pallas as pl
from jax.experimental.pallas import tpu as pltpu
from jax.experimental.pallas import tpu_sc as plsc

VOCAB = 23236
EMBED_DIM = 1728
SEG = 14
PADDED = SEG * 128
BATCH = 1024
SEQ = 50
SEQP = 56
OUT_ROWS = BATCH * (SEQP // 8) * SEG * 8

NC = 2
NS = 16
NW = NC * NS
RPW = OUT_ROWS // NW
CH = 224
NCHUNK = RPW // CH
SUB = 2
NIDX = CH // SUB

_MESH = plsc.VectorSubcoreMesh(core_axis_name="c", subcore_axis_name="s")


@functools.partial(
    pl.kernel,
    out_type=jax.ShapeDtypeStruct((OUT_ROWS, 128), jnp.float32),
    mesh=_MESH,
    compiler_params=pltpu.CompilerParams(use_tc_tiling_on_sc=False),
    scratch_types=[
        pltpu.VMEM((RPW,), jnp.int32),
        pltpu.VMEM((2, CH, 128), jnp.float32),
        pltpu.SemaphoreType.DMA,
        pltpu.SemaphoreType.DMA,
        pltpu.SemaphoreType.DMA,
    ],
)
def _glyph_gather(idx_hbm, tab_hbm, out_hbm, idx_v, rows_v, gsem, osem0, osem1):
    wid = lax.axis_index("s") * NC + lax.axis_index("c")
    base = wid * RPW
    osems = (osem0, osem1)

    pltpu.sync_copy(idx_hbm.at[pl.ds(wid * RPW, RPW)], idx_v)

    def start_gathers(j, b):
        for q in range(SUB):
            pltpu.async_copy(
                tab_hbm.at[idx_v.at[pl.ds(j * CH + q * NIDX, NIDX)]],
                rows_v.at[b, pl.ds(q * NIDX, NIDX)],
                gsem,
            )

    def wait_gathers(b):
        pltpu.make_async_copy(
            tab_hbm.at[pl.ds(0, CH)], rows_v.at[b], gsem
        ).wait()

    start_gathers(0, 0)

    def pair(p, carry):
        for b in range(2):
            j = 2 * p + b
            wait_gathers(b)

            @pl.when(j >= 1)
            def _():
                pltpu.make_async_copy(
                    rows_v.at[1 - b], out_hbm.at[pl.ds(base, CH)], osems[1 - b]
                ).wait()

            @pl.when(j + 1 < NCHUNK)
            def _():
                start_gathers(j + 1, 1 - b)

            pltpu.async_copy(
                rows_v.at[b], out_hbm.at[pl.ds(base + j * CH, CH)], osems[b]
            )
        return carry

    lax.fori_loop(0, NCHUNK // 2, pair, 0)
    pltpu.make_async_copy(
        rows_v.at[1], out_hbm.at[pl.ds(base, CH)], osem1
    ).wait()


def kernel(input_ids, embedding_table):
    ids = input_ids.astype(jnp.int32)
    idsp = jnp.pad(ids, ((0, 0), (0, SEQP - SEQ)))
    segs = (
        idsp.reshape(BATCH, SEQP // 8, 1, 8) * SEG
        + jnp.arange(SEG, dtype=jnp.int32).reshape(1, 1, SEG, 1)
    ).reshape(-1)
    table_seg = jnp.pad(embedding_table, ((0, 0), (0, PADDED - EMBED_DIM)))
    table_seg = table_seg.reshape(VOCAB * SEG, 128)
    out = _glyph_gather(segs, table_seg)
    out = out.reshape(BATCH, SEQP // 8, SEG, 8, 128)
    out = out.transpose(0, 1, 3, 2, 4).reshape(BATCH, SEQP, PADDED)
    one = jnp.minimum(jnp.float32(1), (jnp.abs(ids[0, 0]) + 1).astype(jnp.float32))
    return out[:, :SEQ, :EMBED_DIM] * one
